# Initial kernel scaffold; baseline (speedup 1.0000x reference)
import jax
import jax.numpy as jnp
from jax.experimental import pallas as pl


def kernel(features, id_embedding, edge_index, conv1_W, lin1_W, lin1_b, g1_W, g1_b, conv2_W, lin2_W, lin2_b, g2_W, g2_b):
    raise NotImplementedError("write your pallas kernel here")


# speedup vs baseline: 8.2259x; 8.2259x over previous
import functools

import jax
import jax.numpy as jnp
from jax import lax
from jax.experimental import pallas as pl
from jax.experimental.pallas import tpu as pltpu
from jax.experimental.pallas import tpu_sc as plsc

_N = 10000
_E = 320000
_DF = 128
_DI = 64

_NC = 2
_NS = 16
_NW = _NC * _NS
_CHUNK = 128
_EPW = 10240
_E_PAD = _NW * _EPW
_NCHUNK = _EPW // _CHUNK
_PAD_ROWS = 16
_N_ACC = _N + 112
_RPT = _N_ACC // _NS


def _leaky(v):
    return jnp.where(v >= 0, v, 0.01 * v)


def _make_sc_scatter(d):
    mesh = plsc.VectorSubcoreMesh(core_axis_name="c", subcore_axis_name="s",
                                  num_cores=_NC, num_subcores=_NS)

    @functools.partial(
        pl.kernel,
        out_type=jax.ShapeDtypeStruct((_NC, _N_ACC, d), jnp.float32),
        mesh=mesh,
        compiler_params=pltpu.CompilerParams(use_tc_tiling_on_sc=False),
        scratch_types=[
            pltpu.VMEM((_NCHUNK, _CHUNK), jnp.int32),
            pltpu.VMEM((_NCHUNK, _CHUNK), jnp.int32),
            pltpu.VMEM((_CHUNK, d), jnp.float32),
            pltpu.VMEM_SHARED((_N_ACC, d), jnp.float32),
            pltpu.SemaphoreType.DMA,
        ],
    )
    def sc_scatter(xw_hbm, src_hbm, dst_hbm, zeros_hbm, out_hbm,
                   src_v, dst_v, rows_v, acc_sh, sem):
        c = lax.axis_index("c")
        s = lax.axis_index("s")
        w = c * _NS + s
        pltpu.sync_copy(zeros_hbm.at[pl.ds(s * _RPT, _RPT)],
                        acc_sh.at[pl.ds(s * _RPT, _RPT)])
        pltpu.sync_copy(src_hbm.at[pl.ds(w * _NCHUNK, _NCHUNK)], src_v)
        pltpu.sync_copy(dst_hbm.at[pl.ds(w * _NCHUNK, _NCHUNK)], dst_v)
        plsc.subcore_barrier()

        def body(j, carry):
            pltpu.async_copy(xw_hbm.at[src_v.at[j]], rows_v, sem).wait()
            pltpu.sync_copy(rows_v, acc_sh.at[dst_v.at[j]], add=True)
            return carry

        lax.fori_loop(0, _NCHUNK, body, 0)
        plsc.subcore_barrier()
        pltpu.sync_copy(acc_sh.at[pl.ds(s * _RPT, _RPT)],
                        out_hbm.at[c, pl.ds(s * _RPT, _RPT)])

    return sc_scatter


_sc_scatter_128 = _make_sc_scatter(_DF)
_sc_scatter_64 = _make_sc_scatter(_DI)


_BLK = 1000


def _dot(a, b):
    return jnp.dot(a, b, preferred_element_type=jnp.float32)


def _pre_body(x_ref, id_ref, c1_ref, l1w_ref, l1b_ref, xw_ref, xhat_ref):
    x = x_ref[...]
    nrm = jnp.sqrt(jnp.sum(x * x, axis=1, keepdims=True))
    xn = x / jnp.maximum(nrm, 1e-12)
    xw_ref[...] = _dot(xn, c1_ref[...])
    xhat_ref[...] = _leaky(_dot(xn, l1w_ref[...]) + l1b_ref[...]) + id_ref[...]


def _mid_body(h0_ref, h1_ref, xhat_ref, id_ref, g1w_ref, g1b_ref,
              c2_ref, l2w_ref, l2b_ref, xw2_ref, xhat2_ref):
    h = _leaky(h0_ref[...] + h1_ref[...])
    x2 = _leaky(_dot(h, g1w_ref[...]) + g1b_ref[...] + xhat_ref[...])
    xw2_ref[...] = _dot(x2, c2_ref[...])
    xhat2_ref[...] = _leaky(_dot(x2, l2w_ref[...]) + l2b_ref[...]) + id_ref[...]


def _post_body(h0_ref, h1_ref, xhat2_ref, g2w_ref, g2b_ref, o_ref):
    h = _leaky(h0_ref[...] + h1_ref[...])
    o_ref[...] = _leaky(_dot(h, g2w_ref[...]) + g2b_ref[...] + xhat2_ref[...])


def _row_spec(d):
    return pl.BlockSpec((_BLK, d), lambda i: (i, 0))


def _full_spec(r, c):
    return pl.BlockSpec((r, c), lambda i: (0, 0))


_GRID = _N // _BLK

_pre_call = pl.pallas_call(
    _pre_body,
    grid=(_GRID,),
    in_specs=[_row_spec(_DF), _row_spec(_DI), _full_spec(_DF, _DF),
              _full_spec(_DF, _DI), _full_spec(1, _DI)],
    out_specs=[_row_spec(_DF), _row_spec(_DI)],
    out_shape=[jax.ShapeDtypeStruct((_N, _DF), jnp.float32),
               jax.ShapeDtypeStruct((_N, _DI), jnp.float32)],
)

_mid_call = pl.pallas_call(
    _mid_body,
    grid=(_GRID,),
    in_specs=[_row_spec(_DF), _row_spec(_DF), _row_spec(_DI), _row_spec(_DI),
              _full_spec(_DF, _DI), _full_spec(1, _DI), _full_spec(_DI, _DI),
              _full_spec(_DI, _DI), _full_spec(1, _DI)],
    out_specs=[_row_spec(_DI), _row_spec(_DI)],
    out_shape=[jax.ShapeDtypeStruct((_N, _DI), jnp.float32),
               jax.ShapeDtypeStruct((_N, _DI), jnp.float32)],
)

_post_call = pl.pallas_call(
    _post_body,
    grid=(_GRID,),
    in_specs=[_row_spec(_DI), _row_spec(_DI), _row_spec(_DI),
              _full_spec(_DI, _DI), _full_spec(1, _DI)],
    out_specs=_row_spec(_DI),
    out_shape=jax.ShapeDtypeStruct((_N, _DI), jnp.float32),
)


def kernel(features, id_embedding, edge_index, conv1_W, lin1_W, lin1_b,
           g1_W, g1_b, conv2_W, lin2_W, lin2_b, g2_W, g2_b):
    ei = edge_index.astype(jnp.int32)
    pad = _E_PAD - _E
    ar = jnp.arange(pad, dtype=jnp.int32)
    src = jnp.concatenate([ei[0], ar % _N]).reshape(_NW * _NCHUNK, _CHUNK)
    dst = jnp.concatenate([ei[1], _N + (ar % _PAD_ROWS)]).reshape(
        _NW * _NCHUNK, _CHUNK)
    z128 = jnp.zeros((_N_ACC, _DF), jnp.float32)
    z64 = jnp.zeros((_N_ACC, _DI), jnp.float32)
    l1b = lin1_b.reshape(1, _DI)
    g1b = g1_b.reshape(1, _DI)
    l2b = lin2_b.reshape(1, _DI)
    g2b = g2_b.reshape(1, _DI)

    xw1, xhat1 = _pre_call(features, id_embedding, conv1_W, lin1_W, l1b)
    hp1 = _sc_scatter_128(xw1, src, dst, z128)
    xw2, xhat2 = _mid_call(hp1[0, :_N], hp1[1, :_N], xhat1, id_embedding,
                           g1_W, g1b, conv2_W, lin2_W, l2b)
    hp2 = _sc_scatter_64(xw2, src, dst, z64)
    out = _post_call(hp2[0, :_N], hp2[1, :_N], xhat2, g2_W, g2b)
    return out


# speedup vs baseline: 11.4109x; 1.3872x over previous
import functools

import jax
import jax.numpy as jnp
from jax import lax
from jax.experimental import pallas as pl
from jax.experimental.pallas import tpu as pltpu
from jax.experimental.pallas import tpu_sc as plsc

_N = 10000
_E = 320000
_DF = 128
_DI = 64

_NC = 2
_NS = 16
_NW = _NC * _NS
_CHUNK = 128
_EPW = 10240
_E_PAD = _NW * _EPW
_NCHUNK = _EPW // _CHUNK
_HCH = _NCHUNK // 2
_PAD_ROWS = 16
_N_ACC = _N + 112
_RPT = _N_ACC // _NS


def _leaky(v):
    return jnp.where(v >= 0, v, 0.01 * v)


def _make_sc_scatter(d):
    mesh = plsc.VectorSubcoreMesh(core_axis_name="c", subcore_axis_name="s",
                                  num_cores=_NC, num_subcores=_NS)

    @functools.partial(
        pl.kernel,
        out_type=jax.ShapeDtypeStruct((_NC, _N_ACC, d), jnp.float32),
        mesh=mesh,
        compiler_params=pltpu.CompilerParams(use_tc_tiling_on_sc=False),
        scratch_types=[
            pltpu.VMEM((_HCH, _CHUNK), jnp.int32),
            pltpu.VMEM((_HCH, _CHUNK), jnp.int32),
            pltpu.VMEM((_CHUNK, d), jnp.float32),
            pltpu.VMEM((_CHUNK, d), jnp.float32),
            pltpu.VMEM_SHARED((_N_ACC, d), jnp.float32),
            pltpu.SemaphoreType.DMA,
            pltpu.SemaphoreType.DMA,
        ],
    )
    def sc_scatter(xw_hbm, src_hbm, dst_hbm, zeros_hbm, out_hbm,
                   src_v, dst_v, rows_a, rows_b, acc_sh, sem_a, sem_b):
        c = lax.axis_index("c")
        s = lax.axis_index("s")
        w = c * _NS + s
        pltpu.sync_copy(zeros_hbm.at[pl.ds(s * _RPT, _RPT)],
                        acc_sh.at[pl.ds(s * _RPT, _RPT)])
        plsc.subcore_barrier()

        for p in range(_NCHUNK // _HCH):
            base = w * _NCHUNK + p * _HCH
            pltpu.sync_copy(src_hbm.at[pl.ds(base, _HCH)], src_v)
            pltpu.sync_copy(dst_hbm.at[pl.ds(base, _HCH)], dst_v)
            pltpu.async_copy(xw_hbm.at[src_v.at[0]], rows_a, sem_a)
            pltpu.async_copy(xw_hbm.at[src_v.at[1]], rows_b, sem_b)

            def body(g, carry):
                j = 2 * g
                pltpu.make_async_copy(xw_hbm.at[src_v.at[0]], rows_a,
                                      sem_a).wait()
                pltpu.sync_copy(rows_a, acc_sh.at[dst_v.at[j]], add=True)
                pltpu.async_copy(xw_hbm.at[src_v.at[j + 2]], rows_a, sem_a)
                pltpu.make_async_copy(xw_hbm.at[src_v.at[0]], rows_b,
                                      sem_b).wait()
                pltpu.sync_copy(rows_b, acc_sh.at[dst_v.at[j + 1]], add=True)
                pltpu.async_copy(xw_hbm.at[src_v.at[j + 3]], rows_b, sem_b)
                return carry

            lax.fori_loop(0, _HCH // 2 - 1, body, 0)
            pltpu.make_async_copy(xw_hbm.at[src_v.at[0]], rows_a, sem_a).wait()
            pltpu.sync_copy(rows_a, acc_sh.at[dst_v.at[_HCH - 2]], add=True)
            pltpu.make_async_copy(xw_hbm.at[src_v.at[0]], rows_b, sem_b).wait()
            pltpu.sync_copy(rows_b, acc_sh.at[dst_v.at[_HCH - 1]], add=True)
        plsc.subcore_barrier()
        pltpu.sync_copy(acc_sh.at[pl.ds(s * _RPT, _RPT)],
                        out_hbm.at[c, pl.ds(s * _RPT, _RPT)])

    return sc_scatter


_sc_scatter_128 = _make_sc_scatter(_DF)
_sc_scatter_64 = _make_sc_scatter(_DI)


_BLK = 1000


def _dot(a, b):
    return jnp.dot(a, b, preferred_element_type=jnp.float32)


def _pre_body(x_ref, id_ref, c1_ref, l1w_ref, l1b_ref, xw_ref, xhat_ref):
    x = x_ref[...]
    nrm = jnp.sqrt(jnp.sum(x * x, axis=1, keepdims=True))
    xn = x / jnp.maximum(nrm, 1e-12)
    xw_ref[...] = _dot(xn, c1_ref[...])
    xhat_ref[...] = _leaky(_dot(xn, l1w_ref[...]) + l1b_ref[...]) + id_ref[...]


def _mid_body(h0_ref, h1_ref, xhat_ref, id_ref, g1w_ref, g1b_ref,
              c2_ref, l2w_ref, l2b_ref, xw2_ref, xhat2_ref):
    h = _leaky(h0_ref[...] + h1_ref[...])
    x2 = _leaky(_dot(h, g1w_ref[...]) + g1b_ref[...] + xhat_ref[...])
    xw2_ref[...] = _dot(x2, c2_ref[...])
    xhat2_ref[...] = _leaky(_dot(x2, l2w_ref[...]) + l2b_ref[...]) + id_ref[...]


def _post_body(h0_ref, h1_ref, xhat2_ref, g2w_ref, g2b_ref, o_ref):
    h = _leaky(h0_ref[...] + h1_ref[...])
    o_ref[...] = _leaky(_dot(h, g2w_ref[...]) + g2b_ref[...] + xhat2_ref[...])


def _row_spec(d):
    return pl.BlockSpec((_BLK, d), lambda i: (i, 0))


def _full_spec(r, c):
    return pl.BlockSpec((r, c), lambda i: (0, 0))


_GRID = _N // _BLK

_pre_call = pl.pallas_call(
    _pre_body,
    grid=(_GRID,),
    in_specs=[_row_spec(_DF), _row_spec(_DI), _full_spec(_DF, _DF),
              _full_spec(_DF, _DI), _full_spec(1, _DI)],
    out_specs=[_row_spec(_DF), _row_spec(_DI)],
    out_shape=[jax.ShapeDtypeStruct((_N, _DF), jnp.float32),
               jax.ShapeDtypeStruct((_N, _DI), jnp.float32)],
)

_mid_call = pl.pallas_call(
    _mid_body,
    grid=(_GRID,),
    in_specs=[_row_spec(_DF), _row_spec(_DF), _row_spec(_DI), _row_spec(_DI),
              _full_spec(_DF, _DI), _full_spec(1, _DI), _full_spec(_DI, _DI),
              _full_spec(_DI, _DI), _full_spec(1, _DI)],
    out_specs=[_row_spec(_DI), _row_spec(_DI)],
    out_shape=[jax.ShapeDtypeStruct((_N, _DI), jnp.float32),
               jax.ShapeDtypeStruct((_N, _DI), jnp.float32)],
)

_post_call = pl.pallas_call(
    _post_body,
    grid=(_GRID,),
    in_specs=[_row_spec(_DI), _row_spec(_DI), _row_spec(_DI),
              _full_spec(_DI, _DI), _full_spec(1, _DI)],
    out_specs=_row_spec(_DI),
    out_shape=jax.ShapeDtypeStruct((_N, _DI), jnp.float32),
)


def kernel(features, id_embedding, edge_index, conv1_W, lin1_W, lin1_b,
           g1_W, g1_b, conv2_W, lin2_W, lin2_b, g2_W, g2_b):
    ei = edge_index.astype(jnp.int32)
    pad = _E_PAD - _E
    ar = jnp.arange(pad, dtype=jnp.int32)
    src = jnp.concatenate([ei[0], ar % _N]).reshape(_NW * _NCHUNK, _CHUNK)
    dst = jnp.concatenate([ei[1], _N + (ar % _PAD_ROWS)]).reshape(
        _NW * _NCHUNK, _CHUNK)
    z128 = jnp.zeros((_N_ACC, _DF), jnp.float32)
    z64 = jnp.zeros((_N_ACC, _DI), jnp.float32)
    l1b = lin1_b.reshape(1, _DI)
    g1b = g1_b.reshape(1, _DI)
    l2b = lin2_b.reshape(1, _DI)
    g2b = g2_b.reshape(1, _DI)

    xw1, xhat1 = _pre_call(features, id_embedding, conv1_W, lin1_W, l1b)
    hp1 = _sc_scatter_128(xw1, src, dst, z128)
    xw2, xhat2 = _mid_call(hp1[0, :_N], hp1[1, :_N], xhat1, id_embedding,
                           g1_W, g1b, conv2_W, lin2_W, l2b)
    hp2 = _sc_scatter_64(xw2, src, dst, z64)
    out = _post_call(hp2[0, :_N], hp2[1, :_N], xhat2, g2_W, g2b)
    return out


# speedup vs baseline: 12.0237x; 1.0537x over previous
import functools

import jax
import jax.numpy as jnp
from jax import lax
from jax.experimental import pallas as pl
from jax.experimental.pallas import tpu as pltpu
from jax.experimental.pallas import tpu_sc as plsc

_N = 10000
_E = 320000
_DF = 128
_DI = 64

_NC = 2
_NS = 16
_NW = _NC * _NS
_CHUNK = 128
_EPW = 10240
_E_PAD = _NW * _EPW
_NCHUNK = _EPW // _CHUNK
_HCH = _NCHUNK // 2
_PAD_ROWS = 16
_N_ACC = _N + 112
_RPT = _N_ACC // _NS


def _leaky(v):
    return jnp.where(v >= 0, v, 0.01 * v)


def _make_sc_scatter(d):
    mesh = plsc.VectorSubcoreMesh(core_axis_name="c", subcore_axis_name="s",
                                  num_cores=_NC, num_subcores=_NS)

    params = (pltpu.CompilerParams(use_tc_tiling_on_sc=False)
              if d % 128 != 0 else None)

    @functools.partial(
        pl.kernel,
        out_type=jax.ShapeDtypeStruct((_NC, _N_ACC, d), jnp.float32),
        mesh=mesh,
        compiler_params=params,
        scratch_types=[
            pltpu.VMEM((_HCH, _CHUNK), jnp.int32),
            pltpu.VMEM((_HCH, _CHUNK), jnp.int32),
            pltpu.VMEM((_CHUNK, d), jnp.float32),
            pltpu.VMEM((_CHUNK, d), jnp.float32),
            pltpu.VMEM_SHARED((_N_ACC, d), jnp.float32),
            pltpu.SemaphoreType.DMA,
            pltpu.SemaphoreType.DMA,
        ],
    )
    def sc_scatter(xw_hbm, src_hbm, dst_hbm, zeros_hbm, out_hbm,
                   src_v, dst_v, rows_a, rows_b, acc_sh, sem_a, sem_b):
        c = lax.axis_index("c")
        s = lax.axis_index("s")
        w = c * _NS + s
        pltpu.sync_copy(zeros_hbm.at[pl.ds(s * _RPT, _RPT)],
                        acc_sh.at[pl.ds(s * _RPT, _RPT)])
        plsc.subcore_barrier()

        for p in range(_NCHUNK // _HCH):
            base = w * _NCHUNK + p * _HCH
            pltpu.sync_copy(src_hbm.at[pl.ds(base, _HCH)], src_v)
            pltpu.sync_copy(dst_hbm.at[pl.ds(base, _HCH)], dst_v)
            pltpu.async_copy(xw_hbm.at[src_v.at[0]], rows_a, sem_a)
            pltpu.async_copy(xw_hbm.at[src_v.at[1]], rows_b, sem_b)

            def body(g, carry):
                j = 2 * g
                pltpu.make_async_copy(xw_hbm.at[src_v.at[0]], rows_a,
                                      sem_a).wait()
                pltpu.sync_copy(rows_a, acc_sh.at[dst_v.at[j]], add=True)
                pltpu.async_copy(xw_hbm.at[src_v.at[j + 2]], rows_a, sem_a)
                pltpu.make_async_copy(xw_hbm.at[src_v.at[0]], rows_b,
                                      sem_b).wait()
                pltpu.sync_copy(rows_b, acc_sh.at[dst_v.at[j + 1]], add=True)
                pltpu.async_copy(xw_hbm.at[src_v.at[j + 3]], rows_b, sem_b)
                return carry

            lax.fori_loop(0, _HCH // 2 - 1, body, 0)
            pltpu.make_async_copy(xw_hbm.at[src_v.at[0]], rows_a, sem_a).wait()
            pltpu.sync_copy(rows_a, acc_sh.at[dst_v.at[_HCH - 2]], add=True)
            pltpu.make_async_copy(xw_hbm.at[src_v.at[0]], rows_b, sem_b).wait()
            pltpu.sync_copy(rows_b, acc_sh.at[dst_v.at[_HCH - 1]], add=True)
        plsc.subcore_barrier()
        pltpu.sync_copy(acc_sh.at[pl.ds(s * _RPT, _RPT)],
                        out_hbm.at[c, pl.ds(s * _RPT, _RPT)])

    return sc_scatter


_sc_scatter_128 = _make_sc_scatter(_DF)
_sc_scatter_64 = _make_sc_scatter(_DI)


_BLK = 1000


def _dot(a, b):
    return jnp.dot(a, b, preferred_element_type=jnp.float32)


def _pre_body(x_ref, id_ref, c1_ref, l1w_ref, l1b_ref, xw_ref, xhat_ref):
    x = x_ref[...]
    nrm = jnp.sqrt(jnp.sum(x * x, axis=1, keepdims=True))
    xn = x / jnp.maximum(nrm, 1e-12)
    xw_ref[...] = _dot(xn, c1_ref[...])
    xhat_ref[...] = _leaky(_dot(xn, l1w_ref[...]) + l1b_ref[...]) + id_ref[...]


def _mid_body(h0_ref, h1_ref, xhat_ref, id_ref, g1w_ref, g1b_ref,
              c2_ref, l2w_ref, l2b_ref, xw2_ref, xhat2_ref):
    h = _leaky(h0_ref[0] + h1_ref[0])
    x2 = _leaky(_dot(h, g1w_ref[...]) + g1b_ref[...] + xhat_ref[...])
    xw2_ref[...] = _dot(x2, c2_ref[...])
    xhat2_ref[...] = _leaky(_dot(x2, l2w_ref[...]) + l2b_ref[...]) + id_ref[...]


def _post_body(h0_ref, h1_ref, xhat2_ref, g2w_ref, g2b_ref, o_ref):
    h = _leaky(h0_ref[0] + h1_ref[0])
    o_ref[...] = _leaky(_dot(h, g2w_ref[...]) + g2b_ref[...] + xhat2_ref[...])


def _row_spec(d):
    return pl.BlockSpec((_BLK, d), lambda i: (i, 0))


def _part_spec(core, d):
    return pl.BlockSpec((1, _BLK, d), lambda i, _c=core: (_c, i, 0))


def _full_spec(r, c):
    return pl.BlockSpec((r, c), lambda i: (0, 0))


_GRID = _N // _BLK

_pre_call = pl.pallas_call(
    _pre_body,
    grid=(_GRID,),
    in_specs=[_row_spec(_DF), _row_spec(_DI), _full_spec(_DF, _DF),
              _full_spec(_DF, _DI), _full_spec(1, _DI)],
    out_specs=[_row_spec(_DF), _row_spec(_DI)],
    out_shape=[jax.ShapeDtypeStruct((_N, _DF), jnp.float32),
               jax.ShapeDtypeStruct((_N, _DI), jnp.float32)],
)

_mid_call = pl.pallas_call(
    _mid_body,
    grid=(_GRID,),
    in_specs=[_part_spec(0, _DF), _part_spec(1, _DF), _row_spec(_DI),
              _row_spec(_DI), _full_spec(_DF, _DI), _full_spec(1, _DI),
              _full_spec(_DI, _DI), _full_spec(_DI, _DI), _full_spec(1, _DI)],
    out_specs=[_row_spec(_DI), _row_spec(_DI)],
    out_shape=[jax.ShapeDtypeStruct((_N, _DI), jnp.float32),
               jax.ShapeDtypeStruct((_N, _DI), jnp.float32)],
)

_post_call = pl.pallas_call(
    _post_body,
    grid=(_GRID,),
    in_specs=[_part_spec(0, _DI), _part_spec(1, _DI), _row_spec(_DI),
              _full_spec(_DI, _DI), _full_spec(1, _DI)],
    out_specs=_row_spec(_DI),
    out_shape=jax.ShapeDtypeStruct((_N, _DI), jnp.float32),
)


def kernel(features, id_embedding, edge_index, conv1_W, lin1_W, lin1_b,
           g1_W, g1_b, conv2_W, lin2_W, lin2_b, g2_W, g2_b):
    ei = edge_index.astype(jnp.int32)
    pad = _E_PAD - _E
    ar = jnp.arange(pad, dtype=jnp.int32)
    src = jnp.concatenate([ei[0], ar % _N]).reshape(_NW * _NCHUNK, _CHUNK)
    dst = jnp.concatenate([ei[1], _N + (ar % _PAD_ROWS)]).reshape(
        _NW * _NCHUNK, _CHUNK)
    z128 = jnp.zeros((_N_ACC, _DF), jnp.float32)
    z64 = jnp.zeros((_N_ACC, _DI), jnp.float32)
    l1b = lin1_b.reshape(1, _DI)
    g1b = g1_b.reshape(1, _DI)
    l2b = lin2_b.reshape(1, _DI)
    g2b = g2_b.reshape(1, _DI)

    xw1, xhat1 = _pre_call(features, id_embedding, conv1_W, lin1_W, l1b)
    hp1 = _sc_scatter_128(xw1, src, dst, z128)
    xw2, xhat2 = _mid_call(hp1, hp1, xhat1, id_embedding,
                           g1_W, g1b, conv2_W, lin2_W, l2b)
    hp2 = _sc_scatter_64(xw2, src, dst, z64)
    out = _post_call(hp2, hp2, xhat2, g2_W, g2b)
    return out
